# baseline (speedup 1.0000x reference)
import functools

import jax
import jax.numpy as jnp
from jax import lax
from jax.experimental import pallas as pl
from jax.experimental.pallas import tpu as pltpu
from jax.experimental.pallas import tpu_sc as plsc


def _capsule_concat_sc(a1, a2, ft):
    N, M, D = a2.shape
    R = 1 + 2 * M
    NW = 32
    C = 8
    n_pairs = N // (2 * C)
    lo_p = n_pairs // NW
    n_hi = n_pairs - lo_p * NW
    assert n_hi * (lo_p + 1) + (NW - n_hi) * lo_p == n_pairs

    mesh = plsc.VectorSubcoreMesh(core_axis_name="c", subcore_axis_name="s")

    @functools.partial(
        pl.kernel,
        mesh=mesh,
        out_type=jax.ShapeDtypeStruct((R, N, D), jnp.float32),
        scratch_types=(
            [pltpu.VMEM((C, M, D), jnp.float32)] * 4
            + [pltpu.SemaphoreType.DMA] * 5
        ),
    )
    def k(a1_hbm, a2_hbm, ft_hbm, q_hbm, *scr):
        sa = scr[0:2]
        sf = scr[2:4]
        isems = scr[4:6]
        osems = scr[6:8]
        asem = scr[8]
        wid = lax.axis_index("s") * 2 + lax.axis_index("c")
        pairs_w = jnp.where(wid < n_hi, lo_p + 1, lo_p)
        base_w = (wid * lo_p + jnp.minimum(wid, n_hi)) * 2 * C
        n_full = 2 * pairs_w

        def a1_copy(cnt):
            return pltpu.make_async_copy(
                a1_hbm.at[pl.ds(base_w, cnt)],
                q_hbm.at[0, pl.ds(base_w, cnt)], asem)

        @pl.when(wid < n_hi)
        def _a1_hi():
            a1_copy((lo_p + 1) * 2 * C).start()

        @pl.when(wid >= n_hi)
        def _a1_lo():
            a1_copy(lo_p * 2 * C).start()

        def in_copies(b, base):
            return (
                pltpu.make_async_copy(
                    a2_hbm.at[pl.ds(base, C)], sa[b], isems[b]),
                pltpu.make_async_copy(
                    ft_hbm.at[pl.ds(base, C)], sf[b], isems[b]),
            )

        def out_copies(b, base):
            cs = []
            for j in range(M):
                cs.append(pltpu.make_async_copy(
                    sa[b].at[:, j], q_hbm.at[1 + j, pl.ds(base, C)], osems[b]))
                cs.append(pltpu.make_async_copy(
                    sf[b].at[:, j], q_hbm.at[1 + M + j, pl.ds(base, C)], osems[b]))
            return cs

        for c in in_copies(0, base_w):
            c.start()

        def body(p, carry):
            i0 = p * 2
            for b in range(2):
                i = i0 + b
                base = base_w + i * C

                @pl.when(i + 1 < n_full)
                def _prefetch(b=b, i=i, base=base):
                    @pl.when(i >= 1)
                    def _drain():
                        for c in out_copies(1 - b, base - C):
                            c.wait()
                    for c in in_copies(1 - b, base + C):
                        c.start()

                for c in in_copies(b, base):
                    c.wait()
                for c in out_copies(b, base):
                    c.start()
            return carry

        lax.fori_loop(0, pairs_w, body, 0)

        for b in range(2):
            for c in out_copies(b, base_w + (n_full - 2 + b) * C):
                c.wait()

        @pl.when(wid < n_hi)
        def _a1_hi_wait():
            a1_copy((lo_p + 1) * 2 * C).wait()

        @pl.when(wid >= n_hi)
        def _a1_lo_wait():
            a1_copy(lo_p * 2 * C).wait()

    return k(a1, a2, ft)


@jax.jit
def kernel(a1, a2, ft):
    q = _capsule_concat_sc(a1, a2, ft)
    return jnp.transpose(q, (1, 0, 2))


# speedup vs baseline: 1.6739x; 1.6739x over previous
import jax
import jax.numpy as jnp
from jax.experimental import pallas as pl
from jax.experimental.pallas import tpu as pltpu


def _body(a1_ref, a2_ref, ft_ref, q_ref):
    Mc = a2_ref.shape[1]
    M = (q_ref.shape[0] - 1) // 2
    k = pl.program_id(1)

    @pl.when(k == 0)
    def _():
        q_ref[0, :, :] = a1_ref[...]

    q_ref[pl.ds(1 + Mc * k, Mc), :, :] = jnp.swapaxes(a2_ref[...], 0, 1)
    q_ref[pl.ds(1 + M + Mc * k, Mc), :, :] = jnp.swapaxes(ft_ref[...], 0, 1)


@jax.jit
def kernel(a1, a2, ft):
    N, M, D = a2.shape
    R = 1 + 2 * M
    Bn = 1000
    Mc = 8
    assert N % Bn == 0 and M % Mc == 0
    q = pl.pallas_call(
        _body,
        grid=(N // Bn, M // Mc),
        in_specs=[
            pl.BlockSpec((Bn, D), lambda i, k: (i, 0)),
            pl.BlockSpec((Bn, Mc, D), lambda i, k: (i, k, 0)),
            pl.BlockSpec((Bn, Mc, D), lambda i, k: (i, k, 0)),
        ],
        out_specs=pl.BlockSpec((R, Bn, D), lambda i, k: (0, i, 0)),
        out_shape=jax.ShapeDtypeStruct((R, N, D), jnp.float32),
        compiler_params=pltpu.CompilerParams(
            dimension_semantics=("parallel", "arbitrary"),
        ),
    )(a1, a2, ft)
    return jnp.transpose(q, (1, 0, 2))
